# baseline (speedup 1.0000x reference)
import functools

import jax
import jax.numpy as jnp
from jax import lax
from jax.experimental import pallas as pl
from jax.experimental.pallas import tpu as pltpu
from jax.experimental.pallas import tpu_sc as plsc

_BATCH = 1024
_SEQ = 200
_HIDDEN = 128
_NVEC = _HIDDEN // 16
_NBUF = 4

_info = plsc.get_sparse_core_info()
_NC = _info.num_cores
_NS = _info.num_subcores
_NW = _NC * _NS
_B_PER_W = _BATCH // _NW
_NGRP = _B_PER_W // _NBUF


def _row_add_inplace(buf, cond_v, l):
    cvecs = [cond_v[l, pl.ds(c * 16, 16)] for c in range(_NVEC)]

    def s_body(s, _):
        for u in range(2):
            for c in range(_NVEC):
                plsc.addupdate(buf.at[2 * s + u, pl.ds(c * 16, 16)], cvecs[c])
        return 0

    lax.fori_loop(0, _SEQ // 2, s_body, 0, unroll=False)


def _sc_kernel_body(idx_hbm, x_hbm, table_hbm, out_hbm,
                    idx_v, cond_v, bufs, sem_g, sins, souts):
    wid = lax.axis_index("s") * _NC + lax.axis_index("c")
    base = wid * _B_PER_W

    pltpu.sync_copy(idx_hbm.at[pl.ds(base, _B_PER_W)], idx_v)
    pltpu.async_copy(table_hbm.at[idx_v], cond_v, sem_g).wait()

    def g_body(g, _):
        for b in range(_NBUF):
            @pl.when(g > 0)
            def _():
                pltpu.make_async_copy(bufs[b], out_hbm.at[base], souts[b]).wait()
            pltpu.async_copy(x_hbm.at[base + _NBUF * g + b], bufs[b], sins[b])
        for b in range(_NBUF):
            l = _NBUF * g + b
            pltpu.make_async_copy(x_hbm.at[base], bufs[b], sins[b]).wait()
            _row_add_inplace(bufs[b], cond_v, l)
            pltpu.async_copy(bufs[b], out_hbm.at[base + l], souts[b])
        return 0

    lax.fori_loop(0, _NGRP, g_body, 0, unroll=False)

    for b in range(_NBUF):
        pltpu.make_async_copy(bufs[b], out_hbm.at[base], souts[b]).wait()


def _sc_perturb_add(pert_ids, x, emb):
    mesh = plsc.VectorSubcoreMesh(core_axis_name="c", subcore_axis_name="s")

    def body(idx_hbm, x_hbm, table_hbm, out_hbm,
             idx_v, cond_v, b0, b1, b2, b3,
             sem_g, si0, si1, si2, si3, so0, so1, so2, so3):
        _sc_kernel_body(idx_hbm, x_hbm, table_hbm, out_hbm,
                        idx_v, cond_v, (b0, b1, b2, b3),
                        sem_g, (si0, si1, si2, si3), (so0, so1, so2, so3))

    return functools.partial(
        pl.kernel,
        mesh=mesh,
        out_type=jax.ShapeDtypeStruct((_BATCH, _SEQ, _HIDDEN), jnp.float32),
        scratch_types=[
            pltpu.VMEM((_B_PER_W,), jnp.int32),
            pltpu.VMEM((_B_PER_W, _HIDDEN), jnp.float32),
            pltpu.VMEM((_SEQ, _HIDDEN), jnp.float32),
            pltpu.VMEM((_SEQ, _HIDDEN), jnp.float32),
            pltpu.VMEM((_SEQ, _HIDDEN), jnp.float32),
            pltpu.VMEM((_SEQ, _HIDDEN), jnp.float32),
        ] + [pltpu.SemaphoreType.DMA] * 9,
    )(body)(pert_ids, x, emb)


def kernel(x, pert_ids, emb):
    return _sc_perturb_add(pert_ids.astype(jnp.int32), x, emb)


# speedup vs baseline: 1.1259x; 1.1259x over previous
import functools

import jax
import jax.numpy as jnp
from jax import lax
from jax.experimental import pallas as pl
from jax.experimental.pallas import tpu as pltpu
from jax.experimental.pallas import tpu_sc as plsc

_BATCH = 1024
_SEQ = 200
_HIDDEN = 128

_info = plsc.get_sparse_core_info()
_NC = _info.num_cores
_NS = _info.num_subcores
_NW = _NC * _NS
_B_PER_W = _BATCH // _NW


def _sc_gather(pert_ids, emb):
    mesh = plsc.VectorSubcoreMesh(core_axis_name="c", subcore_axis_name="s")

    half = _B_PER_W // 2

    @functools.partial(
        pl.kernel,
        mesh=mesh,
        out_type=jax.ShapeDtypeStruct((_BATCH, _HIDDEN), jnp.float32),
        scratch_types=[
            pltpu.VMEM((half,), jnp.int32),
            pltpu.VMEM((half,), jnp.int32),
            pltpu.VMEM((half, _HIDDEN), jnp.float32),
            pltpu.VMEM((half, _HIDDEN), jnp.float32),
            pltpu.SemaphoreType.DMA,
            pltpu.SemaphoreType.DMA,
            pltpu.SemaphoreType.DMA,
            pltpu.SemaphoreType.DMA,
            pltpu.SemaphoreType.DMA,
        ],
    )
    def gather_kernel(idx_hbm, table_hbm, out_hbm,
                      idx_a, idx_b, rows_a, rows_b,
                      sia, sib, sga, sgb, swa):
        wid = lax.axis_index("s") * _NC + lax.axis_index("c")
        base = wid * _B_PER_W
        pltpu.async_copy(idx_hbm.at[pl.ds(base, half)], idx_a, sia)
        pltpu.async_copy(idx_hbm.at[pl.ds(base + half, half)], idx_b, sib)
        pltpu.make_async_copy(idx_hbm.at[pl.ds(base, half)], idx_a, sia).wait()
        pltpu.async_copy(table_hbm.at[idx_a], rows_a, sga)
        pltpu.make_async_copy(idx_hbm.at[pl.ds(base, half)], idx_b, sib).wait()
        pltpu.make_async_copy(table_hbm.at[idx_a], rows_a, sga).wait()
        pltpu.async_copy(table_hbm.at[idx_b], rows_b, sgb)
        pltpu.async_copy(rows_a, out_hbm.at[pl.ds(base, half)], swa)
        pltpu.make_async_copy(table_hbm.at[idx_b], rows_b, sgb).wait()
        pltpu.sync_copy(rows_b, out_hbm.at[pl.ds(base + half, half)])
        pltpu.make_async_copy(rows_a, out_hbm.at[pl.ds(base, half)], swa).wait()

    return gather_kernel(pert_ids, emb)


def _add_kernel(x_ref, cond_ref, o_ref):
    o_ref[...] = x_ref[...] + cond_ref[...][:, None, :]


def _tc_broadcast_add(x, cond, bb=128):
    return pl.pallas_call(
        _add_kernel,
        grid=(_BATCH // bb,),
        in_specs=[
            pl.BlockSpec((bb, _SEQ, _HIDDEN), lambda i: (i, 0, 0)),
            pl.BlockSpec((bb, _HIDDEN), lambda i: (i, 0)),
        ],
        out_specs=pl.BlockSpec((bb, _SEQ, _HIDDEN), lambda i: (i, 0, 0)),
        out_shape=jax.ShapeDtypeStruct((_BATCH, _SEQ, _HIDDEN), jnp.float32),
    )(x, cond)


def kernel(x, pert_ids, emb):
    cond = _sc_gather(pert_ids.astype(jnp.int32), emb)
    return _tc_broadcast_add(x, cond)


# speedup vs baseline: 1.1325x; 1.0058x over previous
import functools

import jax
import jax.numpy as jnp
from jax import lax
from jax.experimental import pallas as pl
from jax.experimental.pallas import tpu as pltpu
from jax.experimental.pallas import tpu_sc as plsc

_BATCH = 1024
_SEQ = 200
_HIDDEN = 128

_info = plsc.get_sparse_core_info()
_NC = _info.num_cores
_NS = _info.num_subcores
_NW = _NC * _NS
_B_PER_W = _BATCH // _NW


def _sc_gather(pert_ids, emb):
    mesh = plsc.VectorSubcoreMesh(core_axis_name="c", subcore_axis_name="s")

    @functools.partial(
        pl.kernel,
        mesh=mesh,
        out_type=jax.ShapeDtypeStruct((_BATCH, _HIDDEN), jnp.float32),
        scratch_types=[
            pltpu.VMEM((_B_PER_W,), jnp.int32),
            pltpu.VMEM((_B_PER_W, _HIDDEN), jnp.float32),
            pltpu.SemaphoreType.DMA,
        ],
    )
    def gather_kernel(idx_hbm, table_hbm, out_hbm, idx_v, rows_v, sem):
        wid = lax.axis_index("s") * _NC + lax.axis_index("c")
        base = wid * _B_PER_W
        pltpu.sync_copy(idx_hbm.at[pl.ds(base, _B_PER_W)], idx_v)
        pltpu.async_copy(table_hbm.at[idx_v], rows_v, sem).wait()
        pltpu.sync_copy(rows_v, out_hbm.at[pl.ds(base, _B_PER_W)])

    return gather_kernel(pert_ids, emb)


def _make_add_kernel(bb):
    def _add_kernel(x_ref, cond_ref, o_ref):
        i = pl.program_id(0)
        c = cond_ref[pl.ds(i * bb, bb), :]
        o_ref[...] = x_ref[...] + c[:, None, :]
    return _add_kernel


def _tc_broadcast_add(x, cond, bb=128):
    return pl.pallas_call(
        _make_add_kernel(bb),
        grid=(_BATCH // bb,),
        in_specs=[
            pl.BlockSpec((bb, _SEQ, _HIDDEN), lambda i: (i, 0, 0)),
            pl.BlockSpec((_BATCH, _HIDDEN), lambda i: (0, 0)),
        ],
        out_specs=pl.BlockSpec((bb, _SEQ, _HIDDEN), lambda i: (i, 0, 0)),
        out_shape=jax.ShapeDtypeStruct((_BATCH, _SEQ, _HIDDEN), jnp.float32),
        compiler_params=pltpu.CompilerParams(
            dimension_semantics=("parallel",),
        ),
    )(x, cond)


def kernel(x, pert_ids, emb):
    cond = _sc_gather(pert_ids.astype(jnp.int32), emb)
    return _tc_broadcast_add(x, cond)
